# Initial kernel scaffold; baseline (speedup 1.0000x reference)
import jax
import jax.numpy as jnp
from jax.experimental import pallas as pl


def kernel(x, edge_index, W1, b1, Wl, bl, Wr, W2, b2):
    raise NotImplementedError("write your pallas kernel here")


# speedup vs baseline: 22.4169x; 22.4169x over previous
import functools

import jax
import jax.numpy as jnp
from jax import lax
from jax.experimental import pallas as pl
from jax.experimental.pallas import tpu as pltpu
from jax.experimental.pallas import tpu_sc as plsc

N = 100000
E = 3200000
DI = 16
DH = 32

NC = 2
NS = 16
NW = NC * NS
J = 4
K = J * 128
C = 196
ET = K * C
PE = NW * ET
RPT = 6272
RSTG = 392
NACC = NS * RPT

_f32 = jnp.float32


def _sc_aggregate(h, src2, dst2):
    mesh = plsc.VectorSubcoreMesh(core_axis_name="c", subcore_axis_name="s")

    @functools.partial(
        pl.kernel,
        out_type=(
            jax.ShapeDtypeStruct((NC * NACC, DI), _f32),
            jax.ShapeDtypeStruct((NC * NACC,), _f32),
        ),
        mesh=mesh,
        compiler_params=pltpu.CompilerParams(use_tc_tiling_on_sc=False),
        scratch_types=[
            pltpu.VMEM((J, 128), jnp.int32),
            pltpu.VMEM((J, 128), jnp.int32),
            pltpu.VMEM((K, DI), _f32),
            pltpu.VMEM((128,), _f32),
            pltpu.VMEM((RPT,), _f32),
            pltpu.SemaphoreType.DMA,
            pltpu.VMEM_SHARED((NACC, DI), _f32),
            pltpu.VMEM_SHARED((NACC,), _f32),
        ],
    )
    def agg(h_hbm, src_hbm, dst_hbm, acc_out, cnt_out,
            srcv, dstv, rows, ones, cstg, sem, acc_sh, cnt_sh):
        c = lax.axis_index("c")
        s = lax.axis_index("s")
        wid = c * NS + s

        z16v = jnp.zeros((16,), _f32)
        for k in range(8):
            ones[pl.ds(k * 16, 16)] = jnp.ones((16,), _f32)
        for i in range(RSTG):
            rows[i, :] = z16v
        for i in range(RPT // 16):
            cstg[pl.ds(i * 16, 16)] = z16v

        stg = rows.at[pl.ds(0, RSTG)]
        for t in range(RPT // RSTG):
            pltpu.sync_copy(stg, acc_sh.at[pl.ds(s * RPT + t * RSTG, RSTG)])
        pltpu.sync_copy(cstg, cnt_sh.at[pl.ds(s * RPT, RPT)])
        plsc.subcore_barrier()

        tile_row0 = wid * (ET // 128)

        def chunk(i, carry):
            roff = tile_row0 + i * J
            pltpu.sync_copy(src_hbm.at[pl.ds(roff, J)], srcv)
            pltpu.sync_copy(dst_hbm.at[pl.ds(roff, J)], dstv)
            cps = [
                pltpu.async_copy(
                    h_hbm.at[srcv.at[j]], rows.at[pl.ds(j * 128, 128)], sem)
                for j in range(J)
            ]
            for cp in cps:
                cp.wait()
            for j in range(J):
                pltpu.sync_copy(
                    rows.at[pl.ds(j * 128, 128)], acc_sh.at[dstv.at[j]],
                    add=True)
                pltpu.sync_copy(ones, cnt_sh.at[dstv.at[j]], add=True)
            return carry

        lax.fori_loop(0, C, chunk, 0)
        plsc.subcore_barrier()

        obase = c * NACC + s * RPT
        ostg = rows.at[pl.ds(0, RSTG)]
        for t in range(RPT // RSTG):
            pltpu.sync_copy(acc_sh.at[pl.ds(s * RPT + t * RSTG, RSTG)], ostg)
            pltpu.sync_copy(ostg, acc_out.at[pl.ds(obase + t * RSTG, RSTG)])
        pltpu.sync_copy(cnt_sh.at[pl.ds(s * RPT, RPT)], cstg)
        pltpu.sync_copy(cstg, cnt_out.at[pl.ds(obase, RPT)])

    return agg(h, src2, dst2)


def _tc_h(x, W1, b1):
    R = 4000
    G = N // R

    def body(x_ref, w_ref, b_ref, o_ref):
        o_ref[...] = jnp.maximum(
            jnp.dot(x_ref[...], w_ref[...], preferred_element_type=_f32)
            + b_ref[...], 0.0)

    return pl.pallas_call(
        body,
        grid=(G,),
        in_specs=[
            pl.BlockSpec((R, DI), lambda i: (i, 0)),
            pl.BlockSpec((DI, DI), lambda i: (0, 0)),
            pl.BlockSpec((1, DI), lambda i: (0, 0)),
        ],
        out_specs=pl.BlockSpec((R, DI), lambda i: (i, 0)),
        out_shape=jax.ShapeDtypeStruct((N, DI), _f32),
    )(x, W1, b1.reshape(1, DI))


def _tc_out(p0, p1, c0, c1, h, Wl, bl, Wr, W2, b2):
    R = 4000
    G = N // R

    def body(p0_ref, p1_ref, c0_ref, c1_ref, h_ref, wl_ref, bl_ref, wr_ref,
             w2_ref, b2_ref, o_ref):
        cnt = jnp.maximum(c0_ref[...] + c1_ref[...], 1.0)
        mean = (p0_ref[...] + p1_ref[...]) / cnt
        h2 = jnp.maximum(
            jnp.dot(mean, wl_ref[...], preferred_element_type=_f32)
            + bl_ref[...]
            + jnp.dot(h_ref[...], wr_ref[...], preferred_element_type=_f32),
            0.0)
        o_ref[...] = (
            jnp.dot(h2, w2_ref[...], preferred_element_type=_f32)
            + b2_ref[...])

    return pl.pallas_call(
        body,
        grid=(G,),
        in_specs=[
            pl.BlockSpec((R, DI), lambda i: (i, 0)),
            pl.BlockSpec((R, DI), lambda i: (i, 0)),
            pl.BlockSpec((R, 1), lambda i: (i, 0)),
            pl.BlockSpec((R, 1), lambda i: (i, 0)),
            pl.BlockSpec((R, DI), lambda i: (i, 0)),
            pl.BlockSpec((DI, DH), lambda i: (0, 0)),
            pl.BlockSpec((1, DH), lambda i: (0, 0)),
            pl.BlockSpec((DI, DH), lambda i: (0, 0)),
            pl.BlockSpec((DH, DH), lambda i: (0, 0)),
            pl.BlockSpec((1, DH), lambda i: (0, 0)),
        ],
        out_specs=pl.BlockSpec((R, DH), lambda i: (i, 0)),
        out_shape=jax.ShapeDtypeStruct((N, DH), _f32),
    )(p0, p1, c0, c1, h, Wl, bl.reshape(1, DH), Wr, W2, b2.reshape(1, DH))


def kernel(x, edge_index, W1, b1, Wl, bl, Wr, W2, b2):
    h = _tc_h(x, W1, b1)

    pad = PE - E
    srcp = jnp.concatenate([edge_index[0], jnp.zeros((pad,), jnp.int32)])
    dstp = jnp.concatenate([edge_index[1], jnp.full((pad,), N, jnp.int32)])
    src2 = srcp.reshape(PE // 128, 128)
    dst2 = dstp.reshape(PE // 128, 128)

    acc, cnt = _sc_aggregate(h, src2, dst2)

    p0 = acc[:N]
    p1 = acc[NACC:NACC + N]
    c0 = cnt[:N].reshape(N, 1)
    c1 = cnt[NACC:NACC + N].reshape(N, 1)
    return _tc_out(p0, p1, c0, c1, h, Wl, bl, Wr, W2, b2)


# speedup vs baseline: 31.3706x; 1.3994x over previous
import functools

import jax
import jax.numpy as jnp
from jax import lax
from jax.experimental import pallas as pl
from jax.experimental.pallas import tpu as pltpu
from jax.experimental.pallas import tpu_sc as plsc

N = 100000
E = 3200000
DI = 16
DH = 32

NC = 2
NS = 16
NW = NC * NS
J = 4
K = J * 128
C = 196
ET = K * C
PE = NW * ET
RPT = 6272
RSTG = 392
CSTG = 784
NACC = NS * RPT

_f32 = jnp.float32


def _sc_aggregate(h, src2, dst2):
    mesh = plsc.VectorSubcoreMesh(core_axis_name="c", subcore_axis_name="s")

    @functools.partial(
        pl.kernel,
        out_type=(
            jax.ShapeDtypeStruct((NC * NACC, DI), _f32),
            jax.ShapeDtypeStruct((NC * NACC,), _f32),
        ),
        mesh=mesh,
        compiler_params=pltpu.CompilerParams(use_tc_tiling_on_sc=False),
        scratch_types=[
            pltpu.VMEM((2, J, 128), jnp.int32),
            pltpu.VMEM((2, J, 128), jnp.int32),
            pltpu.VMEM((2, K, DI), _f32),
            pltpu.VMEM((128,), _f32),
            pltpu.VMEM((CSTG,), _f32),
            pltpu.SemaphoreType.DMA,
            pltpu.SemaphoreType.DMA,
            pltpu.VMEM_SHARED((NACC, DI), _f32),
            pltpu.VMEM_SHARED((NACC,), _f32),
        ],
    )
    def agg(h_hbm, src_hbm, dst_hbm, acc_out, cnt_out,
            srcv, dstv, rows, ones, cstg, gsem, ssem, acc_sh, cnt_sh):
        c = lax.axis_index("c")
        s = lax.axis_index("s")
        wid = c * NS + s

        z16v = jnp.zeros((16,), _f32)
        for k in range(8):
            ones[pl.ds(k * 16, 16)] = jnp.ones((16,), _f32)
        for i in range(RSTG):
            rows[0, i, :] = z16v
        for i in range(CSTG // 16):
            cstg[pl.ds(i * 16, 16)] = z16v

        stg = rows.at[0, pl.ds(0, RSTG)]
        for t in range(RPT // RSTG):
            pltpu.sync_copy(stg, acc_sh.at[pl.ds(s * RPT + t * RSTG, RSTG)])
        for t in range(RPT // CSTG):
            pltpu.sync_copy(cstg, cnt_sh.at[pl.ds(s * RPT + t * CSTG, CSTG)])
        plsc.subcore_barrier()

        tile_row0 = wid * (ET // 128)

        def load_idx(b, i):
            roff = tile_row0 + i * J
            pltpu.sync_copy(src_hbm.at[pl.ds(roff, J)], srcv.at[b])
            pltpu.sync_copy(dst_hbm.at[pl.ds(roff, J)], dstv.at[b])

        def gathers(b, fire):
            for j in range(J):
                cp = (pltpu.async_copy if fire else pltpu.make_async_copy)(
                    h_hbm.at[srcv.at[b, j]],
                    rows.at[b, pl.ds(j * 128, 128)], gsem)
                if not fire:
                    cp.wait()

        def scatters(b, fire):
            for j in range(J):
                if fire:
                    pltpu.async_copy(
                        rows.at[b, pl.ds(j * 128, 128)],
                        acc_sh.at[dstv.at[b, j]], ssem, add=True)
                    pltpu.async_copy(
                        ones, cnt_sh.at[dstv.at[b, j]], ssem, add=True)
                else:
                    pltpu.make_async_copy(
                        rows.at[b, pl.ds(j * 128, 128)],
                        acc_sh.at[dstv.at[b, j]], ssem).wait()
                    pltpu.make_async_copy(
                        ones, cnt_sh.at[dstv.at[b, j]], ssem).wait()

        load_idx(0, 0)
        gathers(0, True)
        C2 = C // 2

        def body(i2, carry):
            @pl.when(i2 > 0)
            def _():
                scatters(1, False)
            load_idx(1, 2 * i2 + 1)
            gathers(1, True)
            gathers(0, False)
            scatters(0, True)

            @pl.when(i2 < C2 - 1)
            def _():
                load_idx(0, 2 * i2 + 2)
            gathers(1, False)
            scatters(0, False)
            scatters(1, True)

            @pl.when(i2 < C2 - 1)
            def _():
                gathers(0, True)
            return carry

        lax.fori_loop(0, C2, body, 0)
        scatters(1, False)
        plsc.subcore_barrier()

        obase = c * NACC + s * RPT
        ostg = rows.at[0, pl.ds(0, RSTG)]
        for t in range(RPT // RSTG):
            pltpu.sync_copy(acc_sh.at[pl.ds(s * RPT + t * RSTG, RSTG)], ostg)
            pltpu.sync_copy(ostg, acc_out.at[pl.ds(obase + t * RSTG, RSTG)])
        for t in range(RPT // CSTG):
            pltpu.sync_copy(cnt_sh.at[pl.ds(s * RPT + t * CSTG, CSTG)], cstg)
            pltpu.sync_copy(cstg, cnt_out.at[pl.ds(obase + t * CSTG, CSTG)])

    return agg(h, src2, dst2)


def _tc_h(x, W1, b1):
    R = 4000
    G = N // R

    def body(x_ref, w_ref, b_ref, o_ref):
        o_ref[...] = jnp.maximum(
            jnp.dot(x_ref[...], w_ref[...], preferred_element_type=_f32)
            + b_ref[...], 0.0)

    return pl.pallas_call(
        body,
        grid=(G,),
        in_specs=[
            pl.BlockSpec((R, DI), lambda i: (i, 0)),
            pl.BlockSpec((DI, DI), lambda i: (0, 0)),
            pl.BlockSpec((1, DI), lambda i: (0, 0)),
        ],
        out_specs=pl.BlockSpec((R, DI), lambda i: (i, 0)),
        out_shape=jax.ShapeDtypeStruct((N, DI), _f32),
    )(x, W1, b1.reshape(1, DI))


def _tc_out(p0, p1, c0, c1, h, Wl, bl, Wr, W2, b2):
    R = 4000
    G = N // R

    def body(p0_ref, p1_ref, c0_ref, c1_ref, h_ref, wl_ref, bl_ref, wr_ref,
             w2_ref, b2_ref, o_ref):
        cnt = jnp.maximum(c0_ref[...] + c1_ref[...], 1.0)
        mean = (p0_ref[...] + p1_ref[...]) / cnt
        h2 = jnp.maximum(
            jnp.dot(mean, wl_ref[...], preferred_element_type=_f32)
            + bl_ref[...]
            + jnp.dot(h_ref[...], wr_ref[...], preferred_element_type=_f32),
            0.0)
        o_ref[...] = (
            jnp.dot(h2, w2_ref[...], preferred_element_type=_f32)
            + b2_ref[...])

    return pl.pallas_call(
        body,
        grid=(G,),
        in_specs=[
            pl.BlockSpec((R, DI), lambda i: (i, 0)),
            pl.BlockSpec((R, DI), lambda i: (i, 0)),
            pl.BlockSpec((R, 1), lambda i: (i, 0)),
            pl.BlockSpec((R, 1), lambda i: (i, 0)),
            pl.BlockSpec((R, DI), lambda i: (i, 0)),
            pl.BlockSpec((DI, DH), lambda i: (0, 0)),
            pl.BlockSpec((1, DH), lambda i: (0, 0)),
            pl.BlockSpec((DI, DH), lambda i: (0, 0)),
            pl.BlockSpec((DH, DH), lambda i: (0, 0)),
            pl.BlockSpec((1, DH), lambda i: (0, 0)),
        ],
        out_specs=pl.BlockSpec((R, DH), lambda i: (i, 0)),
        out_shape=jax.ShapeDtypeStruct((N, DH), _f32),
    )(p0, p1, c0, c1, h, Wl, bl.reshape(1, DH), Wr, W2, b2.reshape(1, DH))


def kernel(x, edge_index, W1, b1, Wl, bl, Wr, W2, b2):
    h = _tc_h(x, W1, b1)

    pad = PE - E
    srcp = jnp.concatenate([edge_index[0], jnp.zeros((pad,), jnp.int32)])
    dstp = jnp.concatenate([edge_index[1], jnp.full((pad,), N, jnp.int32)])
    src2 = srcp.reshape(PE // 128, 128)
    dst2 = dstp.reshape(PE // 128, 128)

    acc, cnt = _sc_aggregate(h, src2, dst2)

    p0 = acc[:N]
    p1 = acc[NACC:NACC + N]
    c0 = cnt[:N].reshape(N, 1)
    c1 = cnt[NACC:NACC + N].reshape(N, 1)
    return _tc_out(p0, p1, c0, c1, h, Wl, bl, Wr, W2, b2)


# speedup vs baseline: 31.4686x; 1.0031x over previous
import functools

import jax
import jax.numpy as jnp
from jax import lax
from jax.experimental import pallas as pl
from jax.experimental.pallas import tpu as pltpu
from jax.experimental.pallas import tpu_sc as plsc

N = 100000
E = 3200000
DI = 16
DH = 32

NC = 2
NS = 16
NW = NC * NS
J = 4
K = J * 128
C = 196
ET = K * C
PE = NW * ET
RPT = 6272
RSTG = 392
CSTG = 784
NACC = NS * RPT

_f32 = jnp.float32


def _sc_aggregate(h, src2, dst2):
    mesh = plsc.VectorSubcoreMesh(core_axis_name="c", subcore_axis_name="s")

    @functools.partial(
        pl.kernel,
        out_type=(
            jax.ShapeDtypeStruct((NC * NACC, DI), _f32),
            jax.ShapeDtypeStruct((NC * NACC,), _f32),
        ),
        mesh=mesh,
        compiler_params=pltpu.CompilerParams(use_tc_tiling_on_sc=False),
        scratch_types=[
            pltpu.VMEM((2, K), jnp.int32),
            pltpu.VMEM((2, K), jnp.int32),
            pltpu.VMEM((2, K, DI), _f32),
            pltpu.VMEM((K,), _f32),
            pltpu.VMEM((CSTG,), _f32),
            pltpu.SemaphoreType.DMA,
            pltpu.SemaphoreType.DMA,
            pltpu.VMEM_SHARED((NACC, DI), _f32),
            pltpu.VMEM_SHARED((NACC,), _f32),
        ],
    )
    def agg(h_hbm, src_hbm, dst_hbm, acc_out, cnt_out,
            srcv, dstv, rows, ones, cstg, gsem, ssem, acc_sh, cnt_sh):
        c = lax.axis_index("c")
        s = lax.axis_index("s")
        wid = c * NS + s

        z16v = jnp.zeros((16,), _f32)
        for k in range(K // 16):
            ones[pl.ds(k * 16, 16)] = jnp.ones((16,), _f32)
        for i in range(RSTG):
            rows[0, i, :] = z16v
        for i in range(CSTG // 16):
            cstg[pl.ds(i * 16, 16)] = z16v

        stg = rows.at[0, pl.ds(0, RSTG)]
        for t in range(RPT // RSTG):
            pltpu.sync_copy(stg, acc_sh.at[pl.ds(s * RPT + t * RSTG, RSTG)])
        for t in range(RPT // CSTG):
            pltpu.sync_copy(cstg, cnt_sh.at[pl.ds(s * RPT + t * CSTG, CSTG)])
        plsc.subcore_barrier()

        tile_chunk0 = wid * C

        def load_idx(b, i):
            ci = tile_chunk0 + i
            pltpu.sync_copy(src_hbm.at[ci], srcv.at[b])
            pltpu.sync_copy(dst_hbm.at[ci], dstv.at[b])

        def gathers(b, fire):
            cp = (pltpu.async_copy if fire else pltpu.make_async_copy)(
                h_hbm.at[srcv.at[b]], rows.at[b], gsem)
            if not fire:
                cp.wait()

        def scatters(b, fire):
            if fire:
                pltpu.async_copy(
                    rows.at[b], acc_sh.at[dstv.at[b]], ssem, add=True)
                pltpu.async_copy(
                    ones, cnt_sh.at[dstv.at[b]], ssem, add=True)
            else:
                pltpu.make_async_copy(
                    rows.at[b], acc_sh.at[dstv.at[b]], ssem).wait()
                pltpu.make_async_copy(
                    ones, cnt_sh.at[dstv.at[b]], ssem).wait()

        load_idx(0, 0)
        gathers(0, True)
        C2 = C // 2

        def body(i2, carry):
            @pl.when(i2 > 0)
            def _():
                scatters(1, False)
            load_idx(1, 2 * i2 + 1)
            gathers(1, True)
            gathers(0, False)
            scatters(0, True)

            @pl.when(i2 < C2 - 1)
            def _():
                load_idx(0, 2 * i2 + 2)
            gathers(1, False)
            scatters(0, False)
            scatters(1, True)

            @pl.when(i2 < C2 - 1)
            def _():
                gathers(0, True)
            return carry

        lax.fori_loop(0, C2, body, 0)
        scatters(1, False)
        plsc.subcore_barrier()

        obase = c * NACC + s * RPT
        ostg = rows.at[0, pl.ds(0, RSTG)]
        for t in range(RPT // RSTG):
            pltpu.sync_copy(acc_sh.at[pl.ds(s * RPT + t * RSTG, RSTG)], ostg)
            pltpu.sync_copy(ostg, acc_out.at[pl.ds(obase + t * RSTG, RSTG)])
        for t in range(RPT // CSTG):
            pltpu.sync_copy(cnt_sh.at[pl.ds(s * RPT + t * CSTG, CSTG)], cstg)
            pltpu.sync_copy(cstg, cnt_out.at[pl.ds(obase + t * CSTG, CSTG)])

    return agg(h, src2, dst2)


def _tc_h(x, W1, b1):
    R = 4000
    G = N // R

    def body(x_ref, w_ref, b_ref, o_ref):
        o_ref[...] = jnp.maximum(
            jnp.dot(x_ref[...], w_ref[...], preferred_element_type=_f32)
            + b_ref[...], 0.0)

    return pl.pallas_call(
        body,
        grid=(G,),
        in_specs=[
            pl.BlockSpec((R, DI), lambda i: (i, 0)),
            pl.BlockSpec((DI, DI), lambda i: (0, 0)),
            pl.BlockSpec((1, DI), lambda i: (0, 0)),
        ],
        out_specs=pl.BlockSpec((R, DI), lambda i: (i, 0)),
        out_shape=jax.ShapeDtypeStruct((N, DI), _f32),
    )(x, W1, b1.reshape(1, DI))


def _tc_out(p0, p1, c0, c1, h, Wl, bl, Wr, W2, b2):
    R = 4000
    G = N // R

    def body(p0_ref, p1_ref, c0_ref, c1_ref, h_ref, wl_ref, bl_ref, wr_ref,
             w2_ref, b2_ref, o_ref):
        cnt = jnp.maximum(c0_ref[...] + c1_ref[...], 1.0)
        mean = (p0_ref[...] + p1_ref[...]) / cnt
        h2 = jnp.maximum(
            jnp.dot(mean, wl_ref[...], preferred_element_type=_f32)
            + bl_ref[...]
            + jnp.dot(h_ref[...], wr_ref[...], preferred_element_type=_f32),
            0.0)
        o_ref[...] = (
            jnp.dot(h2, w2_ref[...], preferred_element_type=_f32)
            + b2_ref[...])

    return pl.pallas_call(
        body,
        grid=(G,),
        in_specs=[
            pl.BlockSpec((R, DI), lambda i: (i, 0)),
            pl.BlockSpec((R, DI), lambda i: (i, 0)),
            pl.BlockSpec((R, 1), lambda i: (i, 0)),
            pl.BlockSpec((R, 1), lambda i: (i, 0)),
            pl.BlockSpec((R, DI), lambda i: (i, 0)),
            pl.BlockSpec((DI, DH), lambda i: (0, 0)),
            pl.BlockSpec((1, DH), lambda i: (0, 0)),
            pl.BlockSpec((DI, DH), lambda i: (0, 0)),
            pl.BlockSpec((DH, DH), lambda i: (0, 0)),
            pl.BlockSpec((1, DH), lambda i: (0, 0)),
        ],
        out_specs=pl.BlockSpec((R, DH), lambda i: (i, 0)),
        out_shape=jax.ShapeDtypeStruct((N, DH), _f32),
    )(p0, p1, c0, c1, h, Wl, bl.reshape(1, DH), Wr, W2, b2.reshape(1, DH))


def kernel(x, edge_index, W1, b1, Wl, bl, Wr, W2, b2):
    h = _tc_h(x, W1, b1)

    pad = PE - E
    srcp = jnp.concatenate([edge_index[0], jnp.zeros((pad,), jnp.int32)])
    dstp = jnp.concatenate([edge_index[1], jnp.full((pad,), N, jnp.int32)])
    src2 = srcp.reshape(PE // K, K)
    dst2 = dstp.reshape(PE // K, K)

    acc, cnt = _sc_aggregate(h, src2, dst2)

    p0 = acc[:N]
    p1 = acc[NACC:NACC + N]
    c0 = cnt[:N].reshape(N, 1)
    c1 = cnt[NACC:NACC + N].reshape(N, 1)
    return _tc_out(p0, p1, c0, c1, h, Wl, bl, Wr, W2, b2)


# speedup vs baseline: 41.8280x; 1.3292x over previous
import functools

import jax
import jax.numpy as jnp
from jax import lax
from jax.experimental import pallas as pl
from jax.experimental.pallas import tpu as pltpu
from jax.experimental.pallas import tpu_sc as plsc

N = 100000
E = 3200000
DI = 16
DH = 32

NC = 2
NS = 16
NW = NC * NS
K = 512
NCH = E // K
CLO = NCH // NW
NHI = NCH - CLO * NW
RPT = 6272
RSTG = 392
CSTG = 784
NACC = NS * RPT
RPB = NACC // NW

_f32 = jnp.float32


def _sc_aggregate(h, srcp, dstp):
    mesh = plsc.VectorSubcoreMesh(core_axis_name="c", subcore_axis_name="s")

    @functools.partial(
        pl.kernel,
        out_type=(
            jax.ShapeDtypeStruct((NC * NACC, DI), _f32),
            jax.ShapeDtypeStruct((NC * NACC,), _f32),
        ),
        mesh=mesh,
        compiler_params=pltpu.CompilerParams(use_tc_tiling_on_sc=False),
        scratch_types=[
            pltpu.VMEM((2, K), jnp.int32),
            pltpu.VMEM((2, K), jnp.int32),
            pltpu.VMEM((2, K, DI), _f32),
            pltpu.VMEM((K,), _f32),
            pltpu.VMEM((CSTG,), _f32),
            pltpu.SemaphoreType.DMA,
            pltpu.SemaphoreType.DMA,
            pltpu.VMEM_SHARED((NACC, DI), _f32),
            pltpu.VMEM_SHARED((NACC,), _f32),
        ],
    )
    def agg(h_hbm, src_hbm, dst_hbm, acc_out, cnt_out,
            srcv, dstv, rows, ones, cstg, gsem, ssem, acc_sh, cnt_sh):
        c = lax.axis_index("c")
        s = lax.axis_index("s")
        wid = c * NS + s

        z16v = jnp.zeros((16,), _f32)
        for k in range(K // 16):
            ones[pl.ds(k * 16, 16)] = jnp.ones((16,), _f32)
        for i in range(RSTG):
            rows[0, i, :] = z16v
        for i in range(CSTG // 16):
            cstg[pl.ds(i * 16, 16)] = z16v

        stg = rows.at[0, pl.ds(0, RSTG)]
        for t in range(RPT // RSTG):
            pltpu.sync_copy(stg, acc_sh.at[pl.ds(s * RPT + t * RSTG, RSTG)])
        for t in range(RPT // CSTG):
            pltpu.sync_copy(cstg, cnt_sh.at[pl.ds(s * RPT + t * CSTG, CSTG)])
        plsc.subcore_barrier()

        chunk0 = CLO * wid + jnp.minimum(wid, NHI)
        nch = CLO + jnp.where(wid < NHI, 1, 0)

        def load_idx(b, i):
            off = (chunk0 + i) * K
            pltpu.sync_copy(src_hbm.at[pl.ds(off, K)], srcv.at[b])
            pltpu.sync_copy(dst_hbm.at[pl.ds(off, K)], dstv.at[b])

        def gathers(b, fire):
            cp = (pltpu.async_copy if fire else pltpu.make_async_copy)(
                h_hbm.at[srcv.at[b]], rows.at[b], gsem)
            if not fire:
                cp.wait()

        def scatters(b, fire):
            if fire:
                pltpu.async_copy(
                    rows.at[b], acc_sh.at[dstv.at[b]], ssem, add=True)
                pltpu.async_copy(
                    ones, cnt_sh.at[dstv.at[b]], ssem, add=True)
            else:
                pltpu.make_async_copy(
                    rows.at[b], acc_sh.at[dstv.at[b]], ssem).wait()
                pltpu.make_async_copy(
                    ones, cnt_sh.at[dstv.at[b]], ssem).wait()

        load_idx(0, 0)
        gathers(0, True)

        def body(i, carry):
            b = lax.rem(i, 2)
            nb = 1 - b

            @pl.when(i > 0)
            def _():
                scatters(nb, False)

            @pl.when(i + 1 < nch)
            def _():
                load_idx(nb, i + 1)
                gathers(nb, True)

            gathers(b, False)
            scatters(b, True)
            return carry

        lax.fori_loop(0, nch, body, 0)
        scatters(lax.rem(nch - 1, 2), False)
        plsc.subcore_barrier()

        obase = c * NACC + s * RPT
        ostg = rows.at[0, pl.ds(0, RSTG)]
        for t in range(RPT // RSTG):
            pltpu.sync_copy(acc_sh.at[pl.ds(s * RPT + t * RSTG, RSTG)], ostg)
            pltpu.sync_copy(ostg, acc_out.at[pl.ds(obase + t * RSTG, RSTG)])
        for t in range(RPT // CSTG):
            pltpu.sync_copy(cnt_sh.at[pl.ds(s * RPT + t * CSTG, CSTG)], cstg)
            pltpu.sync_copy(cstg, cnt_out.at[pl.ds(obase + t * CSTG, CSTG)])

    return agg(h, srcp, dstp)


def _sc_mean(accf, cnt):
    mesh = plsc.VectorSubcoreMesh(core_axis_name="c", subcore_axis_name="s")

    @functools.partial(
        pl.kernel,
        out_type=jax.ShapeDtypeStruct((NACC * DI,), _f32),
        mesh=mesh,
        compiler_params=pltpu.CompilerParams(use_tc_tiling_on_sc=False),
        scratch_types=[
            pltpu.VMEM((RPB * DI,), _f32),
            pltpu.VMEM((RPB * DI,), _f32),
            pltpu.VMEM((RPB,), _f32),
            pltpu.VMEM((RPB,), _f32),
        ],
    )
    def meank(acc_hbm, cnt_hbm, mean_out, a0, a1, c0, c1):
        c = lax.axis_index("c")
        s = lax.axis_index("s")
        wid = c * NS + s
        r0 = wid * RPB
        pltpu.sync_copy(acc_hbm.at[pl.ds(r0 * DI, RPB * DI)], a0)
        pltpu.sync_copy(acc_hbm.at[pl.ds((NACC + r0) * DI, RPB * DI)], a1)
        pltpu.sync_copy(cnt_hbm.at[pl.ds(r0, RPB)], c0)
        pltpu.sync_copy(cnt_hbm.at[pl.ds(NACC + r0, RPB)], c1)

        one16 = jnp.ones((16,), _f32)

        def grp(g, carry):
            cs = jnp.maximum(c0[pl.ds(g * 16, 16)] + c1[pl.ds(g * 16, 16)],
                             one16)
            inv = one16 / cs
            for j in range(16):
                base = (g * 16 + j) * DI
                row = a0[pl.ds(base, 16)] + a1[pl.ds(base, 16)]
                a0[pl.ds(base, 16)] = row * jnp.broadcast_to(inv[j], (16,))
            return carry

        lax.fori_loop(0, RPB // 16, grp, 0)
        pltpu.sync_copy(a0, mean_out.at[pl.ds(r0 * DI, RPB * DI)])

    return meank(accf, cnt)


def _tc_h(x, W1, b1):
    R = 4000
    G = N // R

    def body(x_ref, w_ref, b_ref, o_ref):
        o_ref[...] = jnp.maximum(
            jnp.dot(x_ref[...], w_ref[...], preferred_element_type=_f32)
            + b_ref[...], 0.0)

    return pl.pallas_call(
        body,
        grid=(G,),
        in_specs=[
            pl.BlockSpec((R, DI), lambda i: (i, 0)),
            pl.BlockSpec((DI, DI), lambda i: (0, 0)),
            pl.BlockSpec((1, DI), lambda i: (0, 0)),
        ],
        out_specs=pl.BlockSpec((R, DI), lambda i: (i, 0)),
        out_shape=jax.ShapeDtypeStruct((N, DI), _f32),
    )(x, W1, b1.reshape(1, DI))


def _tc_out(mean, h, Wl, bl, Wr, W2, b2):
    R = 4000
    G = N // R

    def body(m_ref, h_ref, wl_ref, bl_ref, wr_ref, w2_ref, b2_ref, o_ref):
        h2 = jnp.maximum(
            jnp.dot(m_ref[...], wl_ref[...], preferred_element_type=_f32)
            + bl_ref[...]
            + jnp.dot(h_ref[...], wr_ref[...], preferred_element_type=_f32),
            0.0)
        o_ref[...] = (
            jnp.dot(h2, w2_ref[...], preferred_element_type=_f32)
            + b2_ref[...])

    return pl.pallas_call(
        body,
        grid=(G,),
        in_specs=[
            pl.BlockSpec((R, DI), lambda i: (i, 0)),
            pl.BlockSpec((R, DI), lambda i: (i, 0)),
            pl.BlockSpec((DI, DH), lambda i: (0, 0)),
            pl.BlockSpec((1, DH), lambda i: (0, 0)),
            pl.BlockSpec((DI, DH), lambda i: (0, 0)),
            pl.BlockSpec((DH, DH), lambda i: (0, 0)),
            pl.BlockSpec((1, DH), lambda i: (0, 0)),
        ],
        out_specs=pl.BlockSpec((R, DH), lambda i: (i, 0)),
        out_shape=jax.ShapeDtypeStruct((N, DH), _f32),
    )(mean, h, Wl, bl.reshape(1, DH), Wr, W2, b2.reshape(1, DH))


def kernel(x, edge_index, W1, b1, Wl, bl, Wr, W2, b2):
    h = _tc_h(x, W1, b1)
    acc, cnt = _sc_aggregate(h, edge_index[0], edge_index[1])
    meanf = _sc_mean(acc.reshape(-1), cnt)
    mean = meanf.reshape(NACC, DI)
    return _tc_out(mean, h, Wl, bl, Wr, W2, b2)


# speedup vs baseline: 43.5142x; 1.0403x over previous
import functools

import jax
import jax.numpy as jnp
from jax import lax
from jax.experimental import pallas as pl
from jax.experimental.pallas import tpu as pltpu
from jax.experimental.pallas import tpu_sc as plsc

N = 100000
E = 3200000
DI = 16
DH = 32

NC = 2
NS = 16
NW = NC * NS
K = 512
NCH = E // K
CLO = NCH // NW
NHI = NCH - CLO * NW
RPT = 6272
RSTG = 392
CSTG = 784
NACC = NS * RPT
RPB = NACC // NW

_f32 = jnp.float32


def _sc_aggregate(h, edges):
    mesh = plsc.VectorSubcoreMesh(core_axis_name="c", subcore_axis_name="s")

    @functools.partial(
        pl.kernel,
        out_type=(
            jax.ShapeDtypeStruct((NC * NACC, DI), _f32),
            jax.ShapeDtypeStruct((NC * NACC,), _f32),
        ),
        mesh=mesh,
        compiler_params=pltpu.CompilerParams(use_tc_tiling_on_sc=False),
        scratch_types=[
            pltpu.VMEM((2, K), jnp.int32),
            pltpu.VMEM((2, K), jnp.int32),
            pltpu.VMEM((2, K, DI), _f32),
            pltpu.VMEM((K,), _f32),
            pltpu.VMEM((CSTG,), _f32),
            pltpu.SemaphoreType.DMA,
            pltpu.SemaphoreType.DMA,
            pltpu.VMEM_SHARED((NACC, DI), _f32),
            pltpu.VMEM_SHARED((NACC,), _f32),
        ],
    )
    def agg(h_hbm, e_hbm, acc_out, cnt_out,
            srcv, dstv, rows, ones, cstg, gsem, ssem, acc_sh, cnt_sh):
        c = lax.axis_index("c")
        s = lax.axis_index("s")
        wid = c * NS + s

        z16v = jnp.zeros((16,), _f32)
        for k in range(K // 16):
            ones[pl.ds(k * 16, 16)] = jnp.ones((16,), _f32)
        for i in range(RSTG):
            rows[0, i, :] = z16v
        for i in range(CSTG // 16):
            cstg[pl.ds(i * 16, 16)] = z16v

        stg = rows.at[0, pl.ds(0, RSTG)]
        for t in range(RPT // RSTG):
            pltpu.sync_copy(stg, acc_sh.at[pl.ds(s * RPT + t * RSTG, RSTG)])
        for t in range(RPT // CSTG):
            pltpu.sync_copy(cstg, cnt_sh.at[pl.ds(s * RPT + t * CSTG, CSTG)])
        plsc.subcore_barrier()

        chunk0 = CLO * wid + jnp.minimum(wid, NHI)
        nch = CLO + jnp.where(wid < NHI, 1, 0)

        def load_idx(b, i):
            off = (chunk0 + i) * K
            pltpu.sync_copy(e_hbm.at[0, pl.ds(off, K)], srcv.at[b])
            pltpu.sync_copy(e_hbm.at[1, pl.ds(off, K)], dstv.at[b])

        def gathers(b, fire):
            cp = (pltpu.async_copy if fire else pltpu.make_async_copy)(
                h_hbm.at[srcv.at[b]], rows.at[b], gsem)
            if not fire:
                cp.wait()

        def scatters(b, fire):
            if fire:
                pltpu.async_copy(
                    rows.at[b], acc_sh.at[dstv.at[b]], ssem, add=True)
                pltpu.async_copy(
                    ones, cnt_sh.at[dstv.at[b]], ssem, add=True)
            else:
                pltpu.make_async_copy(
                    rows.at[b], acc_sh.at[dstv.at[b]], ssem).wait()
                pltpu.make_async_copy(
                    ones, cnt_sh.at[dstv.at[b]], ssem).wait()

        load_idx(0, 0)
        gathers(0, True)

        def body(i, carry):
            b = lax.rem(i, 2)
            nb = 1 - b

            @pl.when(i > 0)
            def _():
                scatters(nb, False)

            @pl.when(i + 1 < nch)
            def _():
                load_idx(nb, i + 1)
                gathers(nb, True)

            gathers(b, False)
            scatters(b, True)
            return carry

        lax.fori_loop(0, nch, body, 0)
        scatters(lax.rem(nch - 1, 2), False)
        plsc.subcore_barrier()

        obase = c * NACC + s * RPT
        ostg = rows.at[0, pl.ds(0, RSTG)]
        for t in range(RPT // RSTG):
            pltpu.sync_copy(acc_sh.at[pl.ds(s * RPT + t * RSTG, RSTG)], ostg)
            pltpu.sync_copy(ostg, acc_out.at[pl.ds(obase + t * RSTG, RSTG)])
        for t in range(RPT // CSTG):
            pltpu.sync_copy(cnt_sh.at[pl.ds(s * RPT + t * CSTG, CSTG)], cstg)
            pltpu.sync_copy(cstg, cnt_out.at[pl.ds(obase + t * CSTG, CSTG)])

    return agg(h, edges)


def _sc_mean(acc, cnt):
    mesh = plsc.VectorSubcoreMesh(core_axis_name="c", subcore_axis_name="s")

    @functools.partial(
        pl.kernel,
        out_type=jax.ShapeDtypeStruct((NACC, DI), _f32),
        mesh=mesh,
        compiler_params=pltpu.CompilerParams(use_tc_tiling_on_sc=False),
        scratch_types=[
            pltpu.VMEM((RPB, DI), _f32),
            pltpu.VMEM((RPB, DI), _f32),
            pltpu.VMEM((RPB,), _f32),
            pltpu.VMEM((RPB,), _f32),
        ],
    )
    def meank(acc_hbm, cnt_hbm, mean_out, a0, a1, c0, c1):
        c = lax.axis_index("c")
        s = lax.axis_index("s")
        wid = c * NS + s
        r0 = wid * RPB
        pltpu.sync_copy(acc_hbm.at[pl.ds(r0, RPB)], a0)
        pltpu.sync_copy(acc_hbm.at[pl.ds(NACC + r0, RPB)], a1)
        pltpu.sync_copy(cnt_hbm.at[pl.ds(r0, RPB)], c0)
        pltpu.sync_copy(cnt_hbm.at[pl.ds(NACC + r0, RPB)], c1)

        one16 = jnp.ones((16,), _f32)

        def grp(g, carry):
            cs = jnp.maximum(c0[pl.ds(g * 16, 16)] + c1[pl.ds(g * 16, 16)],
                             one16)
            inv = one16 / cs
            for j in range(16):
                r = g * 16 + j
                row = a0[r, :] + a1[r, :]
                a0[r, :] = row * jnp.broadcast_to(inv[j], (16,))
            return carry

        lax.fori_loop(0, RPB // 16, grp, 0)
        pltpu.sync_copy(a0, mean_out.at[pl.ds(r0, RPB)])

    return meank(acc, cnt)


def _tc_h(x, W1, b1):
    R = 10000
    G = N // R

    def body(x_ref, w_ref, b_ref, o_ref):
        o_ref[...] = jnp.maximum(
            jnp.dot(x_ref[...], w_ref[...], preferred_element_type=_f32)
            + b_ref[...], 0.0)

    return pl.pallas_call(
        body,
        grid=(G,),
        in_specs=[
            pl.BlockSpec((R, DI), lambda i: (i, 0)),
            pl.BlockSpec((DI, DI), lambda i: (0, 0)),
            pl.BlockSpec((1, DI), lambda i: (0, 0)),
        ],
        out_specs=pl.BlockSpec((R, DI), lambda i: (i, 0)),
        out_shape=jax.ShapeDtypeStruct((N, DI), _f32),
    )(x, W1, b1.reshape(1, DI))


def _tc_out(mean, h, Wl, bl, Wr, W2, b2):
    R = 10000
    G = N // R

    def body(m_ref, h_ref, wl_ref, bl_ref, wr_ref, w2_ref, b2_ref, o_ref):
        h2 = jnp.maximum(
            jnp.dot(m_ref[...], wl_ref[...], preferred_element_type=_f32)
            + bl_ref[...]
            + jnp.dot(h_ref[...], wr_ref[...], preferred_element_type=_f32),
            0.0)
        o_ref[...] = (
            jnp.dot(h2, w2_ref[...], preferred_element_type=_f32)
            + b2_ref[...])

    return pl.pallas_call(
        body,
        grid=(G,),
        in_specs=[
            pl.BlockSpec((R, DI), lambda i: (i, 0)),
            pl.BlockSpec((R, DI), lambda i: (i, 0)),
            pl.BlockSpec((DI, DH), lambda i: (0, 0)),
            pl.BlockSpec((1, DH), lambda i: (0, 0)),
            pl.BlockSpec((DI, DH), lambda i: (0, 0)),
            pl.BlockSpec((DH, DH), lambda i: (0, 0)),
            pl.BlockSpec((1, DH), lambda i: (0, 0)),
        ],
        out_specs=pl.BlockSpec((R, DH), lambda i: (i, 0)),
        out_shape=jax.ShapeDtypeStruct((N, DH), _f32),
    )(mean, h, Wl, bl.reshape(1, DH), Wr, W2, b2.reshape(1, DH))


def kernel(x, edge_index, W1, b1, Wl, bl, Wr, W2, b2):
    h = _tc_h(x, W1, b1)
    acc, cnt = _sc_aggregate(h, edge_index)
    mean = _sc_mean(acc, cnt)
    return _tc_out(mean, h, Wl, bl, Wr, W2, b2)


# speedup vs baseline: 48.9481x; 1.1249x over previous
import functools

import jax
import jax.numpy as jnp
from jax import lax
from jax.experimental import pallas as pl
from jax.experimental.pallas import tpu as pltpu
from jax.experimental.pallas import tpu_sc as plsc

N = 100000
E = 3200000
DI = 16
DH = 32

NC = 2
NS = 16
NW = NC * NS
K = 512
NCH = E // K
CLO = NCH // NW
NHI = NCH - CLO * NW
RPT = 6272
RSTG = 392
CSTG = 784
NACC = NS * RPT
RPB = NACC // NW

_f32 = jnp.float32


def _sc_aggregate(h, edges):
    mesh = plsc.VectorSubcoreMesh(core_axis_name="c", subcore_axis_name="s")

    @functools.partial(
        pl.kernel,
        out_type=(
            jax.ShapeDtypeStruct((NC * NACC, DI), _f32),
            jax.ShapeDtypeStruct((NC * NACC,), _f32),
        ),
        mesh=mesh,
        compiler_params=pltpu.CompilerParams(use_tc_tiling_on_sc=False),
        scratch_types=[
            pltpu.VMEM((2, K), jnp.int32),
            pltpu.VMEM((2, K), jnp.int32),
            pltpu.VMEM((2, K, DI), _f32),
            pltpu.VMEM((K,), _f32),
            pltpu.VMEM((CSTG,), _f32),
            pltpu.SemaphoreType.DMA,
            pltpu.SemaphoreType.DMA,
            pltpu.SemaphoreType.DMA,
            pltpu.VMEM_SHARED((NACC, DI), _f32),
            pltpu.VMEM_SHARED((NACC,), _f32),
        ],
    )
    def agg(h_hbm, e_hbm, acc_out, cnt_out,
            srcv, dstv, rows, ones, cstg, gsem, ssem, isem, acc_sh, cnt_sh):
        c = lax.axis_index("c")
        s = lax.axis_index("s")
        wid = c * NS + s

        z16v = jnp.zeros((16,), _f32)
        for k in range(K // 16):
            ones[pl.ds(k * 16, 16)] = jnp.ones((16,), _f32)
        for i in range(RSTG):
            rows[0, i, :] = z16v
        for i in range(CSTG // 16):
            cstg[pl.ds(i * 16, 16)] = z16v

        stg = rows.at[0, pl.ds(0, RSTG)]
        for t in range(RPT // RSTG):
            pltpu.sync_copy(stg, acc_sh.at[pl.ds(s * RPT + t * RSTG, RSTG)])
        for t in range(RPT // CSTG):
            pltpu.sync_copy(cstg, cnt_sh.at[pl.ds(s * RPT + t * CSTG, CSTG)])
        plsc.subcore_barrier()

        chunk0 = CLO * wid + jnp.minimum(wid, NHI)
        nch = CLO + jnp.where(wid < NHI, 1, 0)

        def load_idx(b, i, fire):
            off = (chunk0 + i) * K
            if fire:
                pltpu.async_copy(e_hbm.at[0, pl.ds(off, K)], srcv.at[b], isem)
                pltpu.async_copy(e_hbm.at[1, pl.ds(off, K)], dstv.at[b], isem)
            else:
                pltpu.make_async_copy(
                    e_hbm.at[0, pl.ds(off, K)], srcv.at[b], isem).wait()
                pltpu.make_async_copy(
                    e_hbm.at[1, pl.ds(off, K)], dstv.at[b], isem).wait()

        def gathers(b, fire):
            cp = (pltpu.async_copy if fire else pltpu.make_async_copy)(
                h_hbm.at[srcv.at[b]], rows.at[b], gsem)
            if not fire:
                cp.wait()

        def scatters(b, fire):
            if fire:
                pltpu.async_copy(
                    rows.at[b], acc_sh.at[dstv.at[b]], ssem, add=True)
                pltpu.async_copy(
                    ones, cnt_sh.at[dstv.at[b]], ssem, add=True)
            else:
                pltpu.make_async_copy(
                    rows.at[b], acc_sh.at[dstv.at[b]], ssem).wait()
                pltpu.make_async_copy(
                    ones, cnt_sh.at[dstv.at[b]], ssem).wait()

        load_idx(0, 0, True)
        load_idx(0, 0, False)
        gathers(0, True)

        def body(i, carry):
            b = lax.rem(i, 2)
            nb = 1 - b

            @pl.when(i > 0)
            def _():
                scatters(nb, False)

            @pl.when(i + 1 < nch)
            def _():
                load_idx(nb, i + 1, True)

            gathers(b, False)
            scatters(b, True)

            @pl.when(i + 1 < nch)
            def _():
                load_idx(nb, i + 1, False)
                gathers(nb, True)
            return carry

        lax.fori_loop(0, nch, body, 0)
        scatters(lax.rem(nch - 1, 2), False)
        plsc.subcore_barrier()

        obase = c * NACC + s * RPT
        ostg = rows.at[0, pl.ds(0, RSTG)]
        for t in range(RPT // RSTG):
            pltpu.sync_copy(acc_sh.at[pl.ds(s * RPT + t * RSTG, RSTG)], ostg)
            pltpu.sync_copy(ostg, acc_out.at[pl.ds(obase + t * RSTG, RSTG)])
        for t in range(RPT // CSTG):
            pltpu.sync_copy(cnt_sh.at[pl.ds(s * RPT + t * CSTG, CSTG)], cstg)
            pltpu.sync_copy(cstg, cnt_out.at[pl.ds(obase + t * CSTG, CSTG)])

    return agg(h, edges)


def _sc_mean(acc, cnt):
    mesh = plsc.VectorSubcoreMesh(core_axis_name="c", subcore_axis_name="s")

    @functools.partial(
        pl.kernel,
        out_type=jax.ShapeDtypeStruct((NACC, DI), _f32),
        mesh=mesh,
        compiler_params=pltpu.CompilerParams(use_tc_tiling_on_sc=False),
        scratch_types=[
            pltpu.VMEM((RPB, DI), _f32),
            pltpu.VMEM((RPB, DI), _f32),
            pltpu.VMEM((RPB,), _f32),
            pltpu.VMEM((RPB,), _f32),
        ],
    )
    def meank(acc_hbm, cnt_hbm, mean_out, a0, a1, c0, c1):
        c = lax.axis_index("c")
        s = lax.axis_index("s")
        wid = c * NS + s
        r0 = wid * RPB
        pltpu.sync_copy(acc_hbm.at[pl.ds(r0, RPB)], a0)
        pltpu.sync_copy(acc_hbm.at[pl.ds(NACC + r0, RPB)], a1)
        pltpu.sync_copy(cnt_hbm.at[pl.ds(r0, RPB)], c0)
        pltpu.sync_copy(cnt_hbm.at[pl.ds(NACC + r0, RPB)], c1)

        one16 = jnp.ones((16,), _f32)

        def grp(g, carry):
            cs = jnp.maximum(c0[pl.ds(g * 16, 16)] + c1[pl.ds(g * 16, 16)],
                             one16)
            inv = one16 / cs
            for j in range(16):
                r = g * 16 + j
                row = a0[r, :] + a1[r, :]
                a0[r, :] = row * jnp.broadcast_to(inv[j], (16,))
            return carry

        lax.fori_loop(0, RPB // 16, grp, 0)
        pltpu.sync_copy(a0, mean_out.at[pl.ds(r0, RPB)])

    return meank(acc, cnt)


def _tc_h(x, W1, b1):
    R = 10000
    G = N // R

    def body(x_ref, w_ref, b_ref, o_ref):
        o_ref[...] = jnp.maximum(
            jnp.dot(x_ref[...], w_ref[...], preferred_element_type=_f32)
            + b_ref[...], 0.0)

    return pl.pallas_call(
        body,
        grid=(G,),
        in_specs=[
            pl.BlockSpec((R, DI), lambda i: (i, 0)),
            pl.BlockSpec((DI, DI), lambda i: (0, 0)),
            pl.BlockSpec((1, DI), lambda i: (0, 0)),
        ],
        out_specs=pl.BlockSpec((R, DI), lambda i: (i, 0)),
        out_shape=jax.ShapeDtypeStruct((N, DI), _f32),
    )(x, W1, b1.reshape(1, DI))


def _tc_out(mean, h, Wl, bl, Wr, W2, b2):
    R = 10000
    G = N // R

    def body(m_ref, h_ref, wl_ref, bl_ref, wr_ref, w2_ref, b2_ref, o_ref):
        h2 = jnp.maximum(
            jnp.dot(m_ref[...], wl_ref[...], preferred_element_type=_f32)
            + bl_ref[...]
            + jnp.dot(h_ref[...], wr_ref[...], preferred_element_type=_f32),
            0.0)
        o_ref[...] = (
            jnp.dot(h2, w2_ref[...], preferred_element_type=_f32)
            + b2_ref[...])

    return pl.pallas_call(
        body,
        grid=(G,),
        in_specs=[
            pl.BlockSpec((R, DI), lambda i: (i, 0)),
            pl.BlockSpec((R, DI), lambda i: (i, 0)),
            pl.BlockSpec((DI, DH), lambda i: (0, 0)),
            pl.BlockSpec((1, DH), lambda i: (0, 0)),
            pl.BlockSpec((DI, DH), lambda i: (0, 0)),
            pl.BlockSpec((DH, DH), lambda i: (0, 0)),
            pl.BlockSpec((1, DH), lambda i: (0, 0)),
        ],
        out_specs=pl.BlockSpec((R, DH), lambda i: (i, 0)),
        out_shape=jax.ShapeDtypeStruct((N, DH), _f32),
    )(mean, h, Wl, bl.reshape(1, DH), Wr, W2, b2.reshape(1, DH))


def kernel(x, edge_index, W1, b1, Wl, bl, Wr, W2, b2):
    h = _tc_h(x, W1, b1)
    acc, cnt = _sc_aggregate(h, edge_index)
    mean = _sc_mean(acc, cnt)
    return _tc_out(mean, h, Wl, bl, Wr, W2, b2)
